# Initial kernel scaffold; baseline (speedup 1.0000x reference)
import jax
import jax.numpy as jnp
from jax.experimental import pallas as pl


def kernel(input_ids, weight):
    raise NotImplementedError("write your pallas kernel here")


# speedup vs baseline: 2.9863x; 2.9863x over previous
import functools

import jax
import jax.numpy as jnp
from jax import lax
from jax.experimental import pallas as pl
from jax.experimental.pallas import tpu as pltpu
from jax.experimental.pallas import tpu_sc as plsc

NC, NS = 2, 16
NW = NC * NS
BATCH, SEQ = 4096, 50
B = BATCH * SEQ
D = 128
ROWS_PER_W = B // NW
CHUNK = 128
G = ROWS_PER_W // CHUNK

_mesh = plsc.VectorSubcoreMesh(core_axis_name="c", subcore_axis_name="s")


@functools.partial(
    pl.kernel,
    out_type=jax.ShapeDtypeStruct((B, D), jnp.float32),
    mesh=_mesh,
    scratch_types=[
        pltpu.VMEM((G, CHUNK), jnp.int32),
        pltpu.VMEM((CHUNK, D), jnp.float32),
        pltpu.VMEM((CHUNK, D), jnp.float32),
        pltpu.SemaphoreType.DMA,
        pltpu.SemaphoreType.DMA,
    ],
)
def _embedding_gather(table_hbm, idx_hbm, out_hbm, idx_v, buf0, buf1, sem0, sem1):
    wid = lax.axis_index("s") * NC + lax.axis_index("c")
    obase = wid * ROWS_PER_W
    pltpu.sync_copy(idx_hbm.at[wid], idx_v)

    def chunk(g, buf, sem):
        pltpu.async_copy(table_hbm.at[idx_v.at[g]], buf, sem).wait()
        pltpu.sync_copy(buf, out_hbm.at[pl.ds(obase + g * CHUNK, CHUNK)])

    def body(i, carry):
        chunk(2 * i, buf0, sem0)
        chunk(2 * i + 1, buf1, sem1)
        return carry

    lax.fori_loop(0, G // 2, body, 0)


def kernel(input_ids, weight):
    idx = input_ids.reshape(NW, G, CHUNK).astype(jnp.int32)
    out = _embedding_gather(weight, idx)
    return out.reshape(BATCH, SEQ, D)


# speedup vs baseline: 3.3598x; 1.1251x over previous
import functools

import jax
import jax.numpy as jnp
from jax import lax
from jax.experimental import pallas as pl
from jax.experimental.pallas import tpu as pltpu
from jax.experimental.pallas import tpu_sc as plsc

NC, NS = 2, 16
NW = NC * NS
BATCH, SEQ = 4096, 50
B = BATCH * SEQ
D = 128
ROWS_PER_W = B // NW
CHUNK = 128
G = ROWS_PER_W // CHUNK
NBUF = 5
LEAD = 3

_mesh = plsc.VectorSubcoreMesh(core_axis_name="c", subcore_axis_name="s")


@functools.partial(
    pl.kernel,
    out_type=jax.ShapeDtypeStruct((B, D), jnp.float32),
    mesh=_mesh,
    scratch_types=(
        [pltpu.VMEM((G, CHUNK), jnp.int32)]
        + [pltpu.VMEM((CHUNK, D), jnp.float32) for _ in range(NBUF)]
        + [pltpu.SemaphoreType.DMA for _ in range(2 * NBUF)]
    ),
)
def _embedding_gather(table_hbm, idx_hbm, out_hbm, idx_v, *scratch):
    bufs = scratch[:NBUF]
    gsem = scratch[NBUF:2 * NBUF]
    ssem = scratch[2 * NBUF:]
    wid = lax.axis_index("s") * NC + lax.axis_index("c")
    obase = wid * ROWS_PER_W
    pltpu.sync_copy(idx_hbm.at[wid], idx_v)

    def start_gather(g, b):
        pltpu.make_async_copy(table_hbm.at[idx_v.at[g]], bufs[b], gsem[b]).start()

    def wait_gather(b):
        pltpu.make_async_copy(table_hbm.at[pl.ds(0, CHUNK)], bufs[b], gsem[b]).wait()

    def start_store(g, b):
        pltpu.make_async_copy(
            bufs[b], out_hbm.at[pl.ds(obase + g * CHUNK, CHUNK)], ssem[b]
        ).start()

    def wait_store(b):
        pltpu.make_async_copy(
            bufs[b], out_hbm.at[pl.ds(obase, CHUNK)], ssem[b]
        ).wait()

    for b in range(LEAD):
        start_gather(b, b)

    def slot(g, b):
        wait_gather(b)
        start_store(g, b)
        gp = g + LEAD
        bp = (b + LEAD) % NBUF

        @pl.when(gp < G)
        def _prefetch():
            @pl.when(gp >= NBUF)
            def _drain():
                wait_store(bp)

            start_gather(gp, bp)

    def body(i, carry):
        for b in range(NBUF):
            slot(i * NBUF + b, b)
        return carry

    lax.fori_loop(0, G // NBUF, body, 0)

    for b in range(NBUF):
        wait_store(b)


def kernel(input_ids, weight):
    idx = input_ids.reshape(NW, G, CHUNK).astype(jnp.int32)
    out = _embedding_gather(weight, idx)
    return out.reshape(BATCH, SEQ, D)
